# baseline (speedup 1.0000x reference)
import jax
import jax.numpy as jnp
from jax import lax
from jax.experimental import pallas as pl
from jax.experimental.pallas import tpu as pltpu
from jax.experimental.pallas import tpu_sc as plsc

EPS = 1e-05
CAP = 55.0
NC = 2
NS = 16
NW = NC * NS


def _sc_body(x_hbm, w_hbm, out_hbm, x_v, w_v, o_v, num_level, embed_dim):
    wid = lax.axis_index("s") * NC + lax.axis_index("c")
    bpw = x_v.shape[0]
    base = wid * bpw
    pltpu.sync_copy(x_hbm.at[pl.ds(base, bpw)], x_v)
    pltpu.sync_copy(w_hbm, w_v)
    lane = jnp.arange(16, dtype=jnp.int32)

    def group(g, _):
        xv = x_v[pl.ds(g * 16, 16)]

        def level(l, carry):
            s = carry[0]
            accs = carry[1:]
            cl = l.astype(jnp.float32)
            d = 1.0 / (jnp.abs(xv - cl) + EPS)
            u = jnp.exp(jnp.minimum(d, CAP))
            wbase = jnp.full((16,), l * embed_dim, jnp.int32)
            new = [s + u]
            for k in range(embed_dim):
                wk = plsc.load_gather(w_v, [wbase + k])
                new.append(accs[k] + u * wk)
            return tuple(new)

        init = tuple(jnp.zeros((16,), jnp.float32) for _ in range(embed_dim + 1))
        res = lax.fori_loop(0, num_level, level, init)
        rinv = 1.0 / res[0]
        rows = (g * 16 + lane) * embed_dim
        for k in range(embed_dim):
            plsc.store_scatter(o_v, [rows + k], res[1 + k] * rinv)
        return 0

    lax.fori_loop(0, bpw // 16, group, 0)
    pltpu.sync_copy(o_v, out_hbm.at[pl.ds(base * embed_dim, bpw * embed_dim)])


def kernel(x, emb_weight, centroid):
    batch = x.shape[0]
    num_level, embed_dim = emb_weight.shape
    bpw = batch // NW
    mesh = plsc.VectorSubcoreMesh(core_axis_name="c", subcore_axis_name="s")
    f = pl.kernel(
        lambda *refs: _sc_body(*refs, num_level, embed_dim),
        mesh=mesh,
        compiler_params=pltpu.CompilerParams(needs_layout_passes=False),
        out_type=jax.ShapeDtypeStruct((batch * embed_dim,), jnp.float32),
        scratch_types=[
            pltpu.VMEM((bpw,), jnp.float32),
            pltpu.VMEM((num_level * embed_dim,), jnp.float32),
            pltpu.VMEM((bpw * embed_dim,), jnp.float32),
        ],
    )
    flat = f(x.reshape(batch), emb_weight.reshape(num_level * embed_dim))
    return flat.reshape(batch, embed_dim)


# speedup vs baseline: 2.5393x; 2.5393x over previous
import jax
import jax.numpy as jnp
from jax import lax
from jax.experimental import pallas as pl
from jax.experimental.pallas import tpu as pltpu
from jax.experimental.pallas import tpu_sc as plsc

EPS = 1e-05
LOG2E = 1.4426950408889634
CAP2 = 80.0
CAPE = 55.0
NC = 2
NS = 16
NW = NC * NS
SC_BATCH = 1536


def _tc_body(x_ref, c_ref, wt_ref, o_ref):
    x = x_ref[...]
    c = c_ref[...]
    d = LOG2E / (jnp.abs(x - c) + EPS)
    u = jnp.exp2(jnp.minimum(d, CAP2))
    vs = jnp.dot(wt_ref[...], u, preferred_element_type=jnp.float32)
    embed_dim = vs.shape[0] - 1
    o_ref[...] = vs[:embed_dim, :] * (1.0 / vs[embed_dim:, :])


def _tc_part(x_row, centroid, w_aug_t, block_b):
    batch = x_row.shape[1]
    num_level = centroid.shape[0]
    embed_dim = w_aug_t.shape[0] - 1
    grid = batch // block_b
    out_t = pl.pallas_call(
        _tc_body,
        grid=(grid,),
        in_specs=[
            pl.BlockSpec((1, block_b), lambda i: (0, i)),
            pl.BlockSpec((num_level, 1), lambda i: (0, 0)),
            pl.BlockSpec((embed_dim + 1, num_level), lambda i: (0, 0)),
        ],
        out_specs=pl.BlockSpec((embed_dim, block_b), lambda i: (0, i)),
        out_shape=jax.ShapeDtypeStruct((embed_dim, batch), jnp.float32),
    )(x_row, centroid, w_aug_t)
    return out_t.T


def _sc_body(x_hbm, w_hbm, out_hbm, x_v, w_v, o_v, num_level, embed_dim):
    wid = lax.axis_index("s") * NC + lax.axis_index("c")
    bpw = x_v.shape[0]
    base = wid * bpw
    pltpu.sync_copy(x_hbm.at[pl.ds(base, bpw)], x_v)
    pltpu.sync_copy(w_hbm, w_v)
    lane = jnp.arange(16, dtype=jnp.int32)

    def group(g, _):
        xv = x_v[pl.ds(g * 16, 16)]

        def level(l, carry):
            s = carry[0]
            accs = carry[1:]
            cl = l.astype(jnp.float32)
            d = 1.0 / (jnp.abs(xv - cl) + EPS)
            u = jnp.exp(jnp.minimum(d, CAPE))
            wbase = jnp.full((16,), l * embed_dim, jnp.int32)
            new = [s + u]
            for k in range(embed_dim):
                wk = plsc.load_gather(w_v, [wbase + k])
                new.append(accs[k] + u * wk)
            return tuple(new)

        init = tuple(jnp.zeros((16,), jnp.float32) for _ in range(embed_dim + 1))
        res = lax.fori_loop(0, num_level, level, init)
        rinv = 1.0 / res[0]
        rows = (g * 16 + lane) * embed_dim
        for k in range(embed_dim):
            plsc.store_scatter(o_v, [rows + k], res[1 + k] * rinv)
        return 0

    lax.fori_loop(0, bpw // 16, group, 0)
    pltpu.sync_copy(o_v, out_hbm.at[pl.ds(base * embed_dim, bpw * embed_dim)])


def _sc_part(x_flat, w_flat, num_level, embed_dim):
    batch = x_flat.shape[0]
    bpw = batch // NW
    mesh = plsc.VectorSubcoreMesh(core_axis_name="c", subcore_axis_name="s")
    f = pl.kernel(
        lambda *refs: _sc_body(*refs, num_level, embed_dim),
        mesh=mesh,
        compiler_params=pltpu.CompilerParams(needs_layout_passes=False),
        out_type=jax.ShapeDtypeStruct((batch * embed_dim,), jnp.float32),
        scratch_types=[
            pltpu.VMEM((bpw,), jnp.float32),
            pltpu.VMEM((num_level * embed_dim,), jnp.float32),
            pltpu.VMEM((bpw * embed_dim,), jnp.float32),
        ],
    )
    return f(x_flat, w_flat).reshape(batch, embed_dim)


def kernel(x, emb_weight, centroid):
    batch = x.shape[0]
    num_level, embed_dim = emb_weight.shape
    tc_batch = batch - SC_BATCH
    x_flat = x.reshape(batch)
    w_aug_t = jnp.concatenate(
        [emb_weight.T, jnp.ones((1, num_level), jnp.float32)], axis=0)
    sc_out = _sc_part(x_flat[tc_batch:], emb_weight.reshape(-1),
                      num_level, embed_dim)
    tc_out = _tc_part(x_flat[:tc_batch].reshape(1, tc_batch), centroid,
                      w_aug_t, tc_batch // 4)
    return jnp.concatenate([tc_out, sc_out], axis=0)


# speedup vs baseline: 8.2285x; 3.2404x over previous
import jax
import jax.numpy as jnp
from jax.experimental import pallas as pl

EPS = 1e-05
LOG2E = 1.4426950408889634
CAP = 80.0
BLOCK_B = 2048


def _body(x_ref, c_ref, wt_ref, o_ref):
    x = x_ref[...]
    c = c_ref[...]
    d = LOG2E / (jnp.abs(x - c) + EPS)
    u = jnp.exp2(jnp.minimum(d, CAP))
    vs = jnp.dot(wt_ref[...], u, preferred_element_type=jnp.float32)
    embed_dim = vs.shape[0] - 1
    o_ref[...] = vs[:embed_dim, :] * (1.0 / vs[embed_dim:, :])


def kernel(x, emb_weight, centroid):
    batch = x.shape[0]
    num_level, embed_dim = emb_weight.shape
    x_row = x.reshape(1, batch)
    w_aug_t = jnp.concatenate(
        [emb_weight.T, jnp.ones((1, num_level), jnp.float32)], axis=0)
    grid = batch // BLOCK_B
    out_t = pl.pallas_call(
        _body,
        grid=(grid,),
        in_specs=[
            pl.BlockSpec((1, BLOCK_B), lambda i: (0, i)),
            pl.BlockSpec((num_level, 1), lambda i: (0, 0)),
            pl.BlockSpec((embed_dim + 1, num_level), lambda i: (0, 0)),
        ],
        out_specs=pl.BlockSpec((embed_dim, BLOCK_B), lambda i: (0, i)),
        out_shape=jax.ShapeDtypeStruct((embed_dim, batch), jnp.float32),
    )(x_row, centroid, w_aug_t)
    return out_t.T


# speedup vs baseline: 11.3234x; 1.3761x over previous
import jax
import jax.numpy as jnp
from jax.experimental import pallas as pl

EPS = 1e-05
LOG2E = 1.4426950408889634
CAP = 80.0
BLOCK_B = 8192


def _body(x_ref, c_ref, wt_ref, o_ref):
    x = x_ref[...]
    c = c_ref[...]
    d = LOG2E / (jnp.abs(x - c) + EPS)
    u = jnp.exp2(jnp.minimum(d, CAP))
    vs = jnp.dot(wt_ref[...], u, preferred_element_type=jnp.float32)
    embed_dim = vs.shape[0] - 1
    o_ref[...] = vs[:embed_dim, :] * (1.0 / vs[embed_dim:, :])


def kernel(x, emb_weight, centroid):
    batch = x.shape[0]
    num_level, embed_dim = emb_weight.shape
    x_row = x.reshape(1, batch)
    w_aug_t = jnp.concatenate(
        [emb_weight.T, jnp.ones((1, num_level), jnp.float32)], axis=0)
    grid = batch // BLOCK_B
    out_t = pl.pallas_call(
        _body,
        grid=(grid,),
        in_specs=[
            pl.BlockSpec((1, BLOCK_B), lambda i: (0, i)),
            pl.BlockSpec((num_level, 1), lambda i: (0, 0)),
            pl.BlockSpec((embed_dim + 1, num_level), lambda i: (0, 0)),
        ],
        out_specs=pl.BlockSpec((embed_dim, BLOCK_B), lambda i: (0, i)),
        out_shape=jax.ShapeDtypeStruct((embed_dim, batch), jnp.float32),
    )(x_row, centroid, w_aug_t)
    return out_t.T


# speedup vs baseline: 11.4125x; 1.0079x over previous
import jax
import jax.numpy as jnp
from jax.experimental import pallas as pl

EPS = 1e-05
LOG2E = 1.4426950408889634
CAP = 80.0
BLOCK_B = 16384


def _body(x_ref, c_ref, wt_ref, o_ref):
    x = x_ref[...]
    c = c_ref[...]
    d = LOG2E / (jnp.abs(x - c) + EPS)
    u = jnp.exp2(jnp.minimum(d, CAP))
    vs = jnp.dot(wt_ref[...], u, preferred_element_type=jnp.float32)
    embed_dim = vs.shape[0] - 1
    o_ref[...] = vs[:embed_dim, :] * (1.0 / vs[embed_dim:, :])


def kernel(x, emb_weight, centroid):
    batch = x.shape[0]
    num_level, embed_dim = emb_weight.shape
    x_row = x.reshape(1, batch)
    w_aug_t = jnp.concatenate(
        [emb_weight.T, jnp.ones((1, num_level), jnp.float32)], axis=0)
    grid = batch // BLOCK_B
    out_t = pl.pallas_call(
        _body,
        grid=(grid,),
        in_specs=[
            pl.BlockSpec((1, BLOCK_B), lambda i: (0, i)),
            pl.BlockSpec((num_level, 1), lambda i: (0, 0)),
            pl.BlockSpec((embed_dim + 1, num_level), lambda i: (0, 0)),
        ],
        out_specs=pl.BlockSpec((embed_dim, BLOCK_B), lambda i: (0, i)),
        out_shape=jax.ShapeDtypeStruct((embed_dim, batch), jnp.float32),
    )(x_row, centroid, w_aug_t)
    return out_t.T
